# Initial kernel scaffold; baseline (speedup 1.0000x reference)
import jax
import jax.numpy as jnp
from jax.experimental import pallas as pl


def kernel(nodes, to_neighs, to_neighs_1, u2e, att1_w, att1_b, att2_w, att2_b, att3_w, att3_b):
    raise NotImplementedError("write your pallas kernel here")


# speedup vs baseline: 1.8188x; 1.8188x over previous
import functools

import jax
import jax.numpy as jnp
from jax import lax
from jax.experimental import pallas as pl
from jax.experimental.pallas import tpu as pltpu
from jax.experimental.pallas import tpu_sc as plsc

_NC = 2
_NS = 16
_NW = _NC * _NS
_CH = 112
_D = 128


def _sc_gather(u2e, idx2d, n_rows):
    nch_w = idx2d.shape[0] // _NW
    per_w = nch_w * _CH
    mesh = plsc.VectorSubcoreMesh(core_axis_name="c", subcore_axis_name="s")

    @functools.partial(
        pl.kernel,
        mesh=mesh,
        out_type=jax.ShapeDtypeStruct((n_rows, _D), jnp.float32),
        scratch_types=[
            pltpu.VMEM((nch_w, _CH), jnp.int32),
            pltpu.VMEM((_CH, _D), jnp.float32),
            pltpu.SemaphoreType.DMA,
        ],
    )
    def gather_kernel(u2e_hbm, idx_hbm, out_hbm, idx_v, buf_v, sem):
        wid = lax.axis_index("s") * _NC + lax.axis_index("c")
        pltpu.sync_copy(
            idx_hbm.at[pl.ds(pl.multiple_of(wid * nch_w, 8), nch_w)], idx_v)
        out_base = wid * per_w

        def body(j, carry):
            pltpu.async_copy(u2e_hbm.at[idx_v.at[j]], buf_v, sem).wait()
            pltpu.sync_copy(
                buf_v,
                out_hbm.at[pl.ds(pl.multiple_of(out_base + j * _CH, 8), _CH)])
            return carry

        lax.fori_loop(0, nch_w, body, 0)

    return gather_kernel(u2e, idx2d)


def _att_stage_body(c_ref, d_ref, w1a_ref, w1b_ref, b1_ref, w2_ref, b2_ref,
                    w3_ref, o_ref, *, gb):
    rb = gb * 16
    c = c_ref[...]
    dp = jnp.dot(d_ref[...], w1b_ref[...],
                 preferred_element_type=jnp.float32) + b1_ref[...]
    dpb = jnp.broadcast_to(dp[:, None, :], (gb, 16, _D)).reshape(rb, _D)
    h = jnp.maximum(
        jnp.dot(c, w1a_ref[...], preferred_element_type=jnp.float32) + dpb, 0.0)
    g = jnp.maximum(
        jnp.dot(h, w2_ref[...], preferred_element_type=jnp.float32)
        + b2_ref[...], 0.0)
    g3 = g.reshape(gb, 16, _D)
    s2 = jnp.sum(g3 * w3_ref[...].reshape(1, 1, _D), axis=2)
    m = jnp.max(s2, axis=1, keepdims=True)
    e = jnp.exp(s2 - m)
    w = e / jnp.sum(e, axis=1, keepdims=True)
    c3 = c.reshape(gb, 16, _D)
    o_ref[...] = jnp.sum(c3 * w[:, :, None], axis=1)


def _att_stage(c_rows, d_rows, w1a, w1b, b1, w2, b2, w3row, *, gb=128,
               interpret=False):
    n = c_rows.shape[0]
    g = d_rows.shape[0]
    rb = gb * 16
    grid = (n // rb,)
    full = lambda i: (0, 0)
    return pl.pallas_call(
        functools.partial(_att_stage_body, gb=gb),
        grid=grid,
        in_specs=[
            pl.BlockSpec((rb, _D), lambda i: (i, 0)),
            pl.BlockSpec((gb, _D), lambda i: (i, 0)),
            pl.BlockSpec((_D, _D), full),
            pl.BlockSpec((_D, _D), full),
            pl.BlockSpec((1, _D), full),
            pl.BlockSpec((_D, _D), full),
            pl.BlockSpec((1, _D), full),
            pl.BlockSpec((1, _D), full),
        ],
        out_specs=pl.BlockSpec((gb, _D), lambda i: (i, 0)),
        out_shape=jax.ShapeDtypeStruct((g, _D), jnp.float32),
        interpret=interpret,
    )(c_rows, d_rows, w1a, w1b, b1, w2, b2, w3row)


def kernel(nodes, to_neighs, to_neighs_1, u2e, att1_w, att1_b, att2_w,
           att2_b, att3_w, att3_b):
    b, k = to_neighs.shape
    k2 = to_neighs_1.shape[2]
    n_two = b * k * k2
    n_one = b * k
    n_rows = n_two + n_one + b

    nch = n_rows // _CH
    nch_w = -(-(nch // _NW) // 8) * 8
    n_pad = _NW * nch_w * _CH

    idx_all = jnp.concatenate([
        to_neighs_1.reshape(-1),
        to_neighs.reshape(-1),
        nodes,
        jnp.zeros((n_pad - n_rows,), dtype=nodes.dtype),
    ]).astype(jnp.int32)
    idx2d = idx_all.reshape(n_pad // _CH, _CH)

    gathered = _sc_gather(u2e, idx2d, n_pad)
    c_two = gathered[:n_two]
    d_one = gathered[n_two:n_two + n_one]
    t_seed = gathered[n_two + n_one:n_rows]

    w1a = att1_w[:_D]
    w1b = att1_w[_D:]
    b1 = att1_b.reshape(1, _D)
    b2 = att2_b.reshape(1, _D)
    w3row = att3_w.reshape(1, _D)

    e_u = _att_stage(c_two, d_one, w1a, w1b, b1, att2_w, b2, w3row)
    out = _att_stage(e_u, t_seed, w1a, w1b, b1, att2_w, b2, w3row)
    return out


# speedup vs baseline: 2.2220x; 1.2216x over previous
import functools

import jax
import jax.numpy as jnp
from jax import lax
from jax.experimental import pallas as pl
from jax.experimental.pallas import tpu as pltpu
from jax.experimental.pallas import tpu_sc as plsc

_NC = 2
_NS = 16
_NW = _NC * _NS
_CH = 112
_SK = 4
_D = 128


def _sc_gather(u2e, idx2d, n_chunks):
    nch_w = n_chunks // _NW
    nsk = nch_w // _SK
    npairs = nsk // 2
    assert nch_w % (2 * _SK) == 0 and nch_w % 8 == 0
    mesh = plsc.VectorSubcoreMesh(core_axis_name="c", subcore_axis_name="s")

    @functools.partial(
        pl.kernel,
        mesh=mesh,
        out_type=jax.ShapeDtypeStruct((n_chunks, _CH, _D), jnp.float32),
        scratch_types=[
            pltpu.VMEM((nch_w, _CH), jnp.int32),
            pltpu.VMEM((_SK, _CH, _D), jnp.float32),
            pltpu.VMEM((_SK, _CH, _D), jnp.float32),
            pltpu.SemaphoreType.DMA,
            pltpu.SemaphoreType.DMA,
        ],
    )
    def gather_kernel(u2e_hbm, idx_hbm, out_hbm, idx_v, buf_a, buf_b,
                      sem_a, sem_b):
        wid = lax.axis_index("s") * _NC + lax.axis_index("c")
        pltpu.sync_copy(
            idx_hbm.at[pl.ds(pl.multiple_of(wid * nch_w, 8), nch_w)], idx_v)
        cbase = wid * nch_w

        def fire(s, buf, sem):
            for i in range(_SK):
                pltpu.async_copy(
                    u2e_hbm.at[idx_v.at[s * _SK + i]], buf.at[i], sem)

        def drain(buf, sem):
            for i in range(_SK):
                pltpu.make_async_copy(
                    u2e_hbm.at[idx_v.at[0]], buf.at[i], sem).wait()

        def store(s, buf):
            pltpu.sync_copy(buf, out_hbm.at[pl.ds(cbase + s * _SK, _SK)])

        fire(0, buf_a, sem_a)

        def body(t, carry):
            fire(2 * t + 1, buf_b, sem_b)
            drain(buf_a, sem_a)
            store(2 * t, buf_a)

            @pl.when(t < npairs - 1)
            def _():
                fire(2 * t + 2, buf_a, sem_a)

            drain(buf_b, sem_b)
            store(2 * t + 1, buf_b)
            return carry

        lax.fori_loop(0, npairs, body, 0)

    return gather_kernel(u2e, idx2d)


def _att_stage_body(c_ref, d_ref, w1a_ref, w1b_ref, b1_ref, w2_ref, b2_ref,
                    w3b_ref, mg_ref, o_ref, *, gb):
    rb = gb * 16
    c = c_ref[...]
    dp = jnp.dot(d_ref[...], w1b_ref[...],
                 preferred_element_type=jnp.float32) + b1_ref[...]
    dpb = jnp.broadcast_to(dp[:, None, :], (gb, 16, _D)).reshape(rb, _D)
    h = jnp.maximum(
        jnp.dot(c, w1a_ref[...], preferred_element_type=jnp.float32) + dpb,
        0.0)
    g = jnp.maximum(
        jnp.dot(h, w2_ref[...], preferred_element_type=jnp.float32)
        + b2_ref[...], 0.0)
    p = jnp.dot(g, w3b_ref[...], preferred_element_type=jnp.float32)
    u = jnp.exp(p)
    mg = mg_ref[...]
    z = jnp.dot(mg, u, preferred_element_type=jnp.float32)
    numer = jnp.dot(mg, u * c, preferred_element_type=jnp.float32)
    o_ref[...] = numer / z


def _att_stage(c_rows, d_rows, w1a, w1b, b1, w2, b2, w3b, mg, *, gb=128,
               interpret=False):
    n = c_rows.shape[0]
    g = d_rows.shape[0]
    rb = gb * 16
    grid = (n // rb,)
    full = lambda i: (0, 0)
    return pl.pallas_call(
        functools.partial(_att_stage_body, gb=gb),
        grid=grid,
        in_specs=[
            pl.BlockSpec((rb, _D), lambda i: (i, 0)),
            pl.BlockSpec((gb, _D), lambda i: (i, 0)),
            pl.BlockSpec((_D, _D), full),
            pl.BlockSpec((_D, _D), full),
            pl.BlockSpec((1, _D), full),
            pl.BlockSpec((_D, _D), full),
            pl.BlockSpec((1, _D), full),
            pl.BlockSpec((_D, _D), full),
            pl.BlockSpec((gb, rb), full),
        ],
        out_specs=pl.BlockSpec((gb, _D), lambda i: (i, 0)),
        out_shape=jax.ShapeDtypeStruct((g, _D), jnp.float32),
        interpret=interpret,
    )(c_rows, d_rows, w1a, w1b, b1, w2, b2, w3b, mg)


def kernel(nodes, to_neighs, to_neighs_1, u2e, att1_w, att1_b, att2_w,
           att2_b, att3_w, att3_b):
    b, k = to_neighs.shape
    k2 = to_neighs_1.shape[2]
    n_two = b * k * k2
    n_one = b * k
    n_rows = n_two + n_one + b

    nch = n_rows // _CH
    nch_w = -(-(nch // _NW) // 8) * 8
    n_chunks = _NW * nch_w
    n_pad = n_chunks * _CH

    idx_all = jnp.concatenate([
        to_neighs_1.reshape(-1),
        to_neighs.reshape(-1),
        nodes,
        jnp.zeros((n_pad - n_rows,), dtype=nodes.dtype),
    ]).astype(jnp.int32)
    idx2d = idx_all.reshape(n_chunks, _CH)

    gathered = _sc_gather(u2e, idx2d, n_chunks).reshape(n_pad, _D)
    c_two = gathered[:n_two]
    d_one = gathered[n_two:n_two + n_one]
    t_seed = gathered[n_two + n_one:n_rows]

    w1a = att1_w[:_D]
    w1b = att1_w[_D:]
    b1 = att1_b.reshape(1, _D)
    b2 = att2_b.reshape(1, _D)
    w3b = jnp.broadcast_to(att3_w, (_D, _D))

    gb = 128
    rb = gb * 16
    gi = jnp.arange(gb, dtype=jnp.int32)[:, None]
    ri = jnp.arange(rb, dtype=jnp.int32)[None, :]
    mg = (ri // 16 == gi).astype(jnp.float32)

    e_u = _att_stage(c_two, d_one, w1a, w1b, b1, att2_w, b2, w3b, mg)
    out = _att_stage(e_u, t_seed, w1a, w1b, b1, att2_w, b2, w3b, mg)
    return out
